# baseline (speedup 1.0000x reference)
import functools

import jax
import jax.numpy as jnp
from jax import lax
from jax.experimental import pallas as pl
from jax.experimental.pallas import tpu as pltpu
from jax.experimental.pallas import tpu_sc as plsc

_BE = 2000


def _node_pre_body(x_ref, w1d_ref, w1s_ref, b1_ref, a_ref, b_ref):
    x = x_ref[...]
    a_ref[...] = jnp.dot(x, w1d_ref[...], preferred_element_type=jnp.float32) + b1_ref[...]
    b_ref[...] = jnp.dot(x, w1s_ref[...], preferred_element_type=jnp.float32)


def _node_pre(x, w1d_t, w1s_t, b1):
    n = x.shape[0]
    hdim = w1d_t.shape[1]
    return pl.pallas_call(
        _node_pre_body,
        out_shape=(jax.ShapeDtypeStruct((n, hdim), jnp.float32),
                   jax.ShapeDtypeStruct((n, hdim), jnp.float32)),
    )(x, w1d_t, w1s_t, b1.reshape(1, -1))


def _part_sum(sa_ref, sb_ref, c_ref, n):
    sa = sa_ref[...]
    sb = sb_ref[...]
    cf = c_ref[...]
    s = sa[0:n] + sa[n:] + sb[0:n] + sb[n:]
    c = jnp.maximum(cf[0:n, 0:1] + cf[n:, 0:1], 1.0)
    return s, c


def _node1_body(x_ref, sa_ref, sb_ref, c_ref, pw_ref, pb_ref, w1i_ref, sb1_ref,
                w1j_ref, h_ref, a_ref, b_ref):
    x = x_ref[...]
    s, c = _part_sum(sa_ref, sb_ref, c_ref, x.shape[0])
    hin = x * (1.0 + s / c)
    h = jnp.dot(hin, pw_ref[...], preferred_element_type=jnp.float32) + pb_ref[...]
    h_ref[...] = h
    a_ref[...] = jnp.dot(h, w1i_ref[...], preferred_element_type=jnp.float32) + sb1_ref[...]
    b_ref[...] = jnp.dot(h, w1j_ref[...], preferred_element_type=jnp.float32)


def _node1(x, s, cnt, pw_t, pb, w1i_t, sb1, w1j_t):
    n, d = x.shape
    hdim = pw_t.shape[1]
    return pl.pallas_call(
        _node1_body,
        out_shape=(jax.ShapeDtypeStruct((n, hdim), jnp.float32),
                   jax.ShapeDtypeStruct((n, hdim), jnp.float32),
                   jax.ShapeDtypeStruct((n, hdim), jnp.float32)),
    )(x, s[0], s[1], cnt, pw_t, pb.reshape(1, -1), w1i_t, sb1.reshape(1, -1),
      w1j_t)


def _bn_relu(h, g, b):
    m = jnp.mean(h, axis=0, keepdims=True)
    v = jnp.mean((h - m) ** 2, axis=0, keepdims=True)
    return jnp.maximum((h - m) * jax.lax.rsqrt(v + 1e-5) * g + b, 0.0)


def _node2_body(h_ref, sa_ref, sb_ref, c_ref, g_ref, bb_ref, w1i_ref, sb1_ref,
                w1j_ref, h_out_ref, a_ref, b_ref):
    s, c = _part_sum(sa_ref, sb_ref, c_ref, h_ref.shape[0])
    h = h_ref[...] * (1.0 + s / c)
    hn = _bn_relu(h, g_ref[...], bb_ref[...])
    h_out_ref[...] = hn
    a_ref[...] = jnp.dot(hn, w1i_ref[...], preferred_element_type=jnp.float32) + sb1_ref[...]
    b_ref[...] = jnp.dot(hn, w1j_ref[...], preferred_element_type=jnp.float32)


def _node2(h, s, cnt, bn_g, bn_b, w1i_t, sb1, w1j_t):
    n, hdim = h.shape
    return pl.pallas_call(
        _node2_body,
        out_shape=(jax.ShapeDtypeStruct((n, hdim), jnp.float32),
                   jax.ShapeDtypeStruct((n, hdim), jnp.float32),
                   jax.ShapeDtypeStruct((n, hdim), jnp.float32)),
    )(h, s[0], s[1], cnt, bn_g.reshape(1, -1), bn_b.reshape(1, -1),
      w1i_t, sb1.reshape(1, -1), w1j_t)


def _node3_body(h_ref, sa_ref, sb_ref, c_ref, g_ref, bb_ref,
                w1_ref, b1_ref, g1_ref, bb1_ref,
                w2_ref, b2_ref, g2_ref, bb2_ref,
                w3_ref, b3_ref, out_ref):
    s, c = _part_sum(sa_ref, sb_ref, c_ref, h_ref.shape[0])
    h = h_ref[...] * (1.0 + s / c)
    hn = _bn_relu(h, g_ref[...], bb_ref[...])
    z = jnp.dot(hn, w1_ref[...], preferred_element_type=jnp.float32) + b1_ref[...]
    z = _bn_relu(z, g1_ref[...], bb1_ref[...])
    z = jnp.dot(z, w2_ref[...], preferred_element_type=jnp.float32) + b2_ref[...]
    z = _bn_relu(z, g2_ref[...], bb2_ref[...])
    out_ref[...] = jnp.dot(z, w3_ref[...], preferred_element_type=jnp.float32) + b3_ref[...]


def _node3(h, s, cnt, bn_g, bn_b, clf):
    n = h.shape[0]
    return pl.pallas_call(
        _node3_body,
        out_shape=jax.ShapeDtypeStruct((n, 1), jnp.float32),
    )(h, s[0], s[1], cnt, bn_g.reshape(1, -1), bn_b.reshape(1, -1),
      clf['W1'].T, clf['b1'].reshape(1, -1), clf['bn1_g'].reshape(1, -1), clf['bn1_b'].reshape(1, -1),
      clf['W2'].T, clf['b2'].reshape(1, -1), clf['bn2_g'].reshape(1, -1), clf['bn2_b'].reshape(1, -1),
      clf['W3'].T, clf['b3'].reshape(1, -1))


def _edge1_body(hid, g_ref, d_ref, freq_ref, ph_ref, wrel_ref, w2_ref, b2_ref,
                q_ref):
    g = g_ref[...]
    if hid < g.shape[1]:
        lane = jax.lax.broadcasted_iota(jnp.int32, g.shape, 1)
        g = jnp.where(lane < hid, g, 0.0)
    rel = jnp.cos(d_ref[...] * freq_ref[...] + ph_ref[...])
    hmid = jnp.maximum(
        g + jnp.dot(rel, wrel_ref[...], preferred_element_type=jnp.float32), 0.0)
    p = jnp.tanh(jnp.dot(hmid.astype(jnp.bfloat16),
                         w2_ref[...].astype(jnp.bfloat16),
                         preferred_element_type=jnp.float32) + b2_ref[...])
    q_ref[...] = 2.0 * p - 1.0


def _edge1(gsum, dts2d, freq, phase, wrel_t, w2_t, b2, hid):
    e, hdim = gsum.shape
    dout = w2_t.shape[1]
    t = freq.shape[0]
    grid = e // _BE
    return pl.pallas_call(
        functools.partial(_edge1_body, hid),
        grid=(grid,),
        in_specs=[
            pl.BlockSpec((_BE, hdim), lambda i: (i, 0)),
            pl.BlockSpec((_BE, 1), lambda i: (i, 0)),
            pl.BlockSpec((1, t), lambda i: (0, 0)),
            pl.BlockSpec((1, t), lambda i: (0, 0)),
            pl.BlockSpec((t, hdim), lambda i: (0, 0)),
            pl.BlockSpec((hdim, dout), lambda i: (0, 0)),
            pl.BlockSpec((1, dout), lambda i: (0, 0)),
        ],
        out_specs=pl.BlockSpec((_BE, dout), lambda i: (i, 0)),
        out_shape=jax.ShapeDtypeStruct((e, dout), jnp.float32),
    )(gsum, dts2d, freq.reshape(1, -1), phase.reshape(1, -1), wrel_t, w2_t,
      b2.reshape(1, -1))


def _edge2_body(g_ref, w2_ref, b2_ref, q_ref):
    hmid = jnp.maximum(g_ref[...], 0.0)
    p = jnp.tanh(jnp.dot(hmid.astype(jnp.bfloat16),
                         w2_ref[...].astype(jnp.bfloat16),
                         preferred_element_type=jnp.float32) + b2_ref[...])
    q_ref[...] = 2.0 * p - 1.0


def _edge2(gsum, w2_t, b2):
    e, hdim = gsum.shape
    dout = w2_t.shape[1]
    grid = e // _BE
    return pl.pallas_call(
        _edge2_body,
        grid=(grid,),
        in_specs=[
            pl.BlockSpec((_BE, hdim), lambda i: (i, 0)),
            pl.BlockSpec((hdim, dout), lambda i: (0, 0)),
            pl.BlockSpec((1, dout), lambda i: (0, 0)),
        ],
        out_specs=pl.BlockSpec((_BE, dout), lambda i: (i, 0)),
        out_shape=jax.ShapeDtypeStruct((e, dout), jnp.float32),
    )(gsum, w2_t, b2.reshape(1, -1))


_NC = 2
_NS = 16
_NW = _NC * _NS
_IB = 100


def _sc_mesh():
    return plsc.VectorSubcoreMesh(core_axis_name="c", subcore_axis_name="s",
                                  num_cores=_NC, num_subcores=_NS)


_SC_PARAMS = pltpu.CompilerParams(use_tc_tiling_on_sc=False)


def _batches(ew, ch):
    out, o = [], 0
    while o < ew:
        b = min(ch, ew - o)
        out.append((o, b))
        o += b
    return out


_SB = 624


def _fill_zbuf(zbuf, dh, zr):
    def zrow(r, _):
        for t in range(dh // 16):
            zbuf[r, pl.ds(t * 16, 16)] = jnp.zeros((16,), jnp.float32)
        return 0

    lax.fori_loop(0, zr, zrow, 0)


def _zero_stripe(zbuf, acc, sid, n, zr):
    start = sid * _SB
    for zo, zb in _batches(_SB, zr):
        pltpu.sync_copy(zbuf.at[pl.ds(0, zb)], acc.at[pl.ds(start + zo, zb)])
    tail = n - _NS * _SB

    @pl.when(sid == _NS - 1)
    def _():
        pltpu.sync_copy(zbuf.at[pl.ds(0, tail)], acc.at[pl.ds(_NS * _SB, tail)])


def _copy_out(acc, out_hbm, cid, sid, n):
    start = sid * _SB
    pltpu.sync_copy(acc.at[pl.ds(start, _SB)],
                    out_hbm.at[pl.ds(cid * n + start, _SB)])
    tail = n - _NS * _SB

    @pl.when(sid == _NS - 1)
    def _():
        pltpu.sync_copy(acc.at[pl.ds(_NS * _SB, tail)],
                        out_hbm.at[pl.ds(cid * n + _NS * _SB, tail)])


def _gather_sum(a_tbl, b_tbl, dst1, src1, out_dh=None):
    n, dh = a_tbl.shape
    out_dh = out_dh or dh
    e = dst1.shape[0]
    ew = e // _NW
    ch = 200

    @functools.partial(
        pl.kernel, mesh=_sc_mesh(),
        out_type=jax.ShapeDtypeStruct((e, out_dh), jnp.float32),
        compiler_params=_SC_PARAMS,
        scratch_types=[
            pltpu.VMEM((ew,), jnp.int32),
            pltpu.VMEM((ew,), jnp.int32),
            pltpu.VMEM((ch, dh), jnp.float32),
            pltpu.VMEM((ch, dh), jnp.float32),
            pltpu.VMEM((ch, dh), jnp.float32),
            pltpu.VMEM((ch, dh), jnp.float32),
            pltpu.SemaphoreType.DMA,
            pltpu.SemaphoreType.DMA,
            pltpu.SemaphoreType.DMA,
            pltpu.SemaphoreType.DMA,
        ],
    )
    def k(a_hbm, b_hbm, dst_hbm, src_hbm, out_hbm, idxa, idxb,
          ba0, bb0, ba1, bb1, sg0, sg1, so0, so1):
        wid = lax.axis_index("s") * _NC + lax.axis_index("c")
        pltpu.sync_copy(dst_hbm.at[pl.ds(wid * ew, ew)], idxa)
        pltpu.sync_copy(src_hbm.at[pl.ds(wid * ew, ew)], idxb)

        ba = (ba0, ba1)
        bb = (bb0, bb1)
        sg = (sg0, sg1)
        so = (so0, so1)
        bs = _batches(ew, ch)
        gcps = [None, None]
        ocps = [None, None]

        def start_gather(i):
            o, ne = bs[i]
            s = i % 2
            isl = pl.ds(o, ne)
            esl = pl.ds(0, ne)
            ca = pltpu.async_copy(a_hbm.at[idxa.at[isl]], ba[s].at[esl], sg[s])
            cb = pltpu.async_copy(b_hbm.at[idxb.at[isl]], bb[s].at[esl], sg[s])
            gcps[s] = (ca, cb)

        start_gather(0)
        for i, (o, ne) in enumerate(bs):
            s = i % 2
            ca, cb = gcps[s]
            ca.wait()
            cb.wait()
            if i + 1 < len(bs):
                if ocps[1 - s] is not None:
                    ocps[1 - s].wait()
                    ocps[1 - s] = None
                start_gather(i + 1)

            def row(r, _):
                for t in range(dh // 16):
                    sl = pl.ds(t * 16, 16)
                    ba[s][r, sl] = ba[s][r, sl] + bb[s][r, sl]
                return 0

            lax.fori_loop(0, ne, row, 0)
            esl = pl.ds(0, ne)
            rsl = pl.ds(wid * ew + o, ne)
            if out_dh == dh:
                ocps[s] = pltpu.async_copy(ba[s].at[esl], out_hbm.at[rsl],
                                           so[s])
            else:
                ocps[s] = pltpu.async_copy(ba[s].at[esl],
                                           out_hbm.at[rsl, pl.ds(0, dh)],
                                           so[s])
        for s in range(2):
            if ocps[s] is not None:
                ocps[s].wait()

    return k(a_tbl, b_tbl, dst1, src1)


def _sc_scatter(q, dst1, n, dst_full=None, cs=8):
    e, dh = q.shape
    ew = e // _NW
    zr = 208
    ch = 1600
    cw = dh // cs

    outs = [jax.ShapeDtypeStruct((_NC * n, dh), jnp.float32)]
    scr = [
        pltpu.VMEM((ew,), jnp.int32),
        pltpu.VMEM((ch, cw), jnp.float32),
        pltpu.VMEM((ch, cw), jnp.float32),
        pltpu.VMEM((zr, cw), jnp.float32),
        pltpu.VMEM_SHARED((n, cw), jnp.float32),
        pltpu.SemaphoreType.DMA,
        pltpu.SemaphoreType.DMA,
    ]
    ewf = None
    if dst_full is not None:
        ewf = dst_full.shape[0] // _NW
        outs.append(jax.ShapeDtypeStruct((_NC * n, cw), jnp.float32))
        scr.append(pltpu.VMEM((ewf,), jnp.int32))

    @functools.partial(
        pl.kernel, mesh=_sc_mesh(),
        out_type=tuple(outs) if len(outs) > 1 else outs[0],
        compiler_params=_SC_PARAMS,
        scratch_types=scr,
    )
    def k(*refs):
        if dst_full is None:
            (q_hbm, dst_hbm, out_hbm, idx, qb0, qb1, zbuf, acc,
             sem0, sem1) = refs
            dstf_hbm = cnt_hbm = idxf = None
        else:
            (q_hbm, dst_hbm, dstf_hbm, out_hbm, cnt_hbm, idx, qb0, qb1, zbuf,
             acc, sem0, sem1, idxf) = refs
        cid = lax.axis_index("c")
        sid = lax.axis_index("s")
        wid = sid * _NC + cid
        qb = (qb0, qb1)
        sem = (sem0, sem1)
        pltpu.sync_copy(dst_hbm.at[pl.ds(wid * ew, ew)], idx)
        _fill_zbuf(zbuf, cw, zr)
        for p in range(cs):
            csl = pl.ds(p * cw, cw)
            _zero_stripe(zbuf, acc, sid, n, zr)
            plsc.subcore_barrier()

            bs = _batches(ew, ch)
            cps = [None, None]

            def load(i):
                o, ne = bs[i]
                s = i % 2
                cps[s] = pltpu.async_copy(
                    q_hbm.at[pl.ds(wid * ew + o, ne), csl],
                    qb[s].at[pl.ds(0, ne)], sem[s])

            load(0)
            for i, (o, ne) in enumerate(bs):
                s = i % 2
                cps[s].wait()
                if i + 1 < len(bs):
                    load(i + 1)
                pltpu.sync_copy(qb[s].at[pl.ds(0, ne)],
                                acc.at[idx.at[pl.ds(o, ne)]], add=True)
            plsc.subcore_barrier()
            start = sid * _SB
            pltpu.sync_copy(acc.at[pl.ds(start, _SB)],
                            out_hbm.at[pl.ds(cid * n + start, _SB), csl])
            tail = n - _NS * _SB

            @pl.when(sid == _NS - 1)
            def _():
                pltpu.sync_copy(acc.at[pl.ds(_NS * _SB, tail)],
                                out_hbm.at[pl.ds(cid * n + _NS * _SB, tail),
                                           csl])
            plsc.subcore_barrier()

        if dst_full is not None:
            pltpu.sync_copy(dstf_hbm.at[pl.ds(wid * ewf, ewf)], idxf)

            def fill(r, _):
                for t in range(cw // 16):
                    qb0[r, pl.ds(t * 16, 16)] = jnp.ones((16,), jnp.float32)
                return 0

            lax.fori_loop(0, ch, fill, 0)
            _zero_stripe(zbuf, acc, sid, n, zr)
            plsc.subcore_barrier()
            for o, ne in _batches(ewf, ch):
                pltpu.sync_copy(qb0.at[pl.ds(0, ne)],
                                acc.at[idxf.at[pl.ds(o, ne)]], add=True)
            plsc.subcore_barrier()
            _copy_out(acc, cnt_hbm, cid, sid, n)

    if dst_full is None:
        return k(q, dst1)
    return k(q, dst1, dst_full)


def kernel(x, dts, params, edge_index):
    src = edge_index[0]
    dst = edge_index[1]
    n, d = x.shape
    e = dst.shape[0]
    t = params['basis_freq'].shape[0]

    w1 = params['tmp_W1']
    hid = w1.shape[0]
    hpad = -hid % 128
    w1d_t = w1[:, :d].T
    w1s_t = w1[:, d:2 * d].T
    w1rel_t = jnp.pad(w1[:, 2 * d:].T, ((0, 0), (0, hpad)))
    tmp_b1 = params['tmp_b1']
    tmp_w2_t = jnp.pad(params['tmp_W2'].T, ((0, hpad), (0, 0)))

    eh = e // 2
    dts2d = dts.reshape(-1, 1)
    hv = []
    for i in range(2):
        sl = slice(i * eh, (i + 1) * eh)
        hv.append((dst[sl], src[sl], dts2d[sl]))

    a1, b1t = _node_pre(x, w1d_t, w1s_t, tmp_b1)
    s1 = []
    cnt = None
    for i, (d3, s3_, dt) in enumerate(hv):
        g = _gather_sum(a1, b1t, d3, s3_, out_dh=hid + hpad)
        q = _edge1(g, dt, params['basis_freq'], params['phase'],
                   w1rel_t, tmp_w2_t, params['tmp_b2'], hid)
        if i == 0:
            s, cnt = _sc_scatter(q, d3, n, dst_full=dst, cs=8)
            s1.append(s)
        else:
            s1.append(_sc_scatter(q, d3, n, cs=8))

    smp0, smp1 = params['smp']
    h, a2, b2t = _node1(x, s1, cnt, params['proj_W'].T, params['proj_b'],
                        smp0['W1'][:, :d].T, smp0['b1'], smp0['W1'][:, d:].T)

    s2 = []
    for d3, s3_, _ in hv:
        g = _gather_sum(a2, b2t, d3, s3_)
        q = _edge2(g, smp0['W2'].T, smp0['b2'])
        s2.append(_sc_scatter(q, d3, n))
    h, a3, b3t = _node2(h, s2, cnt, smp0['bn_g'], smp0['bn_b'],
                        smp1['W1'][:, :d].T, smp1['b1'], smp1['W1'][:, d:].T)

    s3 = []
    for d3, s3_, _ in hv:
        g = _gather_sum(a3, b3t, d3, s3_)
        q = _edge2(g, smp1['W2'].T, smp1['b2'])
        s3.append(_sc_scatter(q, d3, n))

    return _node3(h, s3, cnt, smp1['bn_g'], smp1['bn_b'], params['clf'])


# speedup vs baseline: 1.0015x; 1.0015x over previous
import functools

import jax
import jax.numpy as jnp
from jax import lax
from jax.experimental import pallas as pl
from jax.experimental.pallas import tpu as pltpu
from jax.experimental.pallas import tpu_sc as plsc

_BE = 2000


def _node_pre_body(x_ref, w1d_ref, w1s_ref, b1_ref, a_ref, b_ref):
    x = x_ref[...]
    a_ref[...] = jnp.dot(x, w1d_ref[...], preferred_element_type=jnp.float32) + b1_ref[...]
    b_ref[...] = jnp.dot(x, w1s_ref[...], preferred_element_type=jnp.float32)


def _node_pre(x, w1d_t, w1s_t, b1):
    n = x.shape[0]
    hdim = w1d_t.shape[1]
    return pl.pallas_call(
        _node_pre_body,
        out_shape=(jax.ShapeDtypeStruct((n, hdim), jnp.float32),
                   jax.ShapeDtypeStruct((n, hdim), jnp.float32)),
    )(x, w1d_t, w1s_t, b1.reshape(1, -1))


def _part_sum(sa_ref, sb_ref, c_ref, n):
    sa = sa_ref[...]
    sb = sb_ref[...]
    cf = c_ref[...]
    s = sa[0:n] + sa[n:] + sb[0:n] + sb[n:]
    c = jnp.maximum(cf[0:n, 0:1] + cf[n:, 0:1], 1.0)
    return s, c


def _node1_body(x_ref, sa_ref, sb_ref, c_ref, pw_ref, pb_ref, w1i_ref, sb1_ref,
                w1j_ref, h_ref, a_ref, b_ref):
    x = x_ref[...]
    s, c = _part_sum(sa_ref, sb_ref, c_ref, x.shape[0])
    hin = x * (1.0 + s / c)
    h = jnp.dot(hin, pw_ref[...], preferred_element_type=jnp.float32) + pb_ref[...]
    h_ref[...] = h
    a_ref[...] = jnp.dot(h, w1i_ref[...], preferred_element_type=jnp.float32) + sb1_ref[...]
    b_ref[...] = jnp.dot(h, w1j_ref[...], preferred_element_type=jnp.float32)


def _node1(x, s, cnt, pw_t, pb, w1i_t, sb1, w1j_t):
    n, d = x.shape
    hdim = pw_t.shape[1]
    return pl.pallas_call(
        _node1_body,
        out_shape=(jax.ShapeDtypeStruct((n, hdim), jnp.float32),
                   jax.ShapeDtypeStruct((n, hdim), jnp.float32),
                   jax.ShapeDtypeStruct((n, hdim), jnp.float32)),
    )(x, s[0], s[1], cnt, pw_t, pb.reshape(1, -1), w1i_t, sb1.reshape(1, -1),
      w1j_t)


def _bn_relu(h, g, b):
    m = jnp.mean(h, axis=0, keepdims=True)
    v = jnp.mean((h - m) ** 2, axis=0, keepdims=True)
    return jnp.maximum((h - m) * jax.lax.rsqrt(v + 1e-5) * g + b, 0.0)


def _node2_body(h_ref, sa_ref, sb_ref, c_ref, g_ref, bb_ref, w1i_ref, sb1_ref,
                w1j_ref, h_out_ref, a_ref, b_ref):
    s, c = _part_sum(sa_ref, sb_ref, c_ref, h_ref.shape[0])
    h = h_ref[...] * (1.0 + s / c)
    hn = _bn_relu(h, g_ref[...], bb_ref[...])
    h_out_ref[...] = hn
    a_ref[...] = jnp.dot(hn, w1i_ref[...], preferred_element_type=jnp.float32) + sb1_ref[...]
    b_ref[...] = jnp.dot(hn, w1j_ref[...], preferred_element_type=jnp.float32)


def _node2(h, s, cnt, bn_g, bn_b, w1i_t, sb1, w1j_t):
    n, hdim = h.shape
    return pl.pallas_call(
        _node2_body,
        out_shape=(jax.ShapeDtypeStruct((n, hdim), jnp.float32),
                   jax.ShapeDtypeStruct((n, hdim), jnp.float32),
                   jax.ShapeDtypeStruct((n, hdim), jnp.float32)),
    )(h, s[0], s[1], cnt, bn_g.reshape(1, -1), bn_b.reshape(1, -1),
      w1i_t, sb1.reshape(1, -1), w1j_t)


def _node3_body(h_ref, sa_ref, sb_ref, c_ref, g_ref, bb_ref,
                w1_ref, b1_ref, g1_ref, bb1_ref,
                w2_ref, b2_ref, g2_ref, bb2_ref,
                w3_ref, b3_ref, out_ref):
    s, c = _part_sum(sa_ref, sb_ref, c_ref, h_ref.shape[0])
    h = h_ref[...] * (1.0 + s / c)
    hn = _bn_relu(h, g_ref[...], bb_ref[...])
    z = jnp.dot(hn, w1_ref[...], preferred_element_type=jnp.float32) + b1_ref[...]
    z = _bn_relu(z, g1_ref[...], bb1_ref[...])
    z = jnp.dot(z, w2_ref[...], preferred_element_type=jnp.float32) + b2_ref[...]
    z = _bn_relu(z, g2_ref[...], bb2_ref[...])
    out_ref[...] = jnp.dot(z, w3_ref[...], preferred_element_type=jnp.float32) + b3_ref[...]


def _node3(h, s, cnt, bn_g, bn_b, clf):
    n = h.shape[0]
    return pl.pallas_call(
        _node3_body,
        out_shape=jax.ShapeDtypeStruct((n, 1), jnp.float32),
    )(h, s[0], s[1], cnt, bn_g.reshape(1, -1), bn_b.reshape(1, -1),
      clf['W1'].T, clf['b1'].reshape(1, -1), clf['bn1_g'].reshape(1, -1), clf['bn1_b'].reshape(1, -1),
      clf['W2'].T, clf['b2'].reshape(1, -1), clf['bn2_g'].reshape(1, -1), clf['bn2_b'].reshape(1, -1),
      clf['W3'].T, clf['b3'].reshape(1, -1))


def _edge1_body(hid, g_ref, d_ref, freq_ref, ph_ref, wrel_ref, w2_ref, b2_ref,
                q_ref):
    g = g_ref[...]
    if hid < g.shape[1]:
        lane = jax.lax.broadcasted_iota(jnp.int32, g.shape, 1)
        g = jnp.where(lane < hid, g, 0.0)
    rel = jnp.cos(d_ref[...] * freq_ref[...] + ph_ref[...])
    hmid = jnp.maximum(
        g + jnp.dot(rel, wrel_ref[...], preferred_element_type=jnp.float32), 0.0)
    p = jnp.tanh(jnp.dot(hmid.astype(jnp.bfloat16),
                         w2_ref[...].astype(jnp.bfloat16),
                         preferred_element_type=jnp.float32) + b2_ref[...])
    q_ref[...] = 2.0 * p - 1.0


def _edge1(gsum, dts2d, freq, phase, wrel_t, w2_t, b2, hid):
    e, hdim = gsum.shape
    dout = w2_t.shape[1]
    t = freq.shape[0]
    grid = e // _BE
    return pl.pallas_call(
        functools.partial(_edge1_body, hid),
        grid=(grid,),
        in_specs=[
            pl.BlockSpec((_BE, hdim), lambda i: (i, 0)),
            pl.BlockSpec((_BE, 1), lambda i: (i, 0)),
            pl.BlockSpec((1, t), lambda i: (0, 0)),
            pl.BlockSpec((1, t), lambda i: (0, 0)),
            pl.BlockSpec((t, hdim), lambda i: (0, 0)),
            pl.BlockSpec((hdim, dout), lambda i: (0, 0)),
            pl.BlockSpec((1, dout), lambda i: (0, 0)),
        ],
        out_specs=pl.BlockSpec((_BE, dout), lambda i: (i, 0)),
        out_shape=jax.ShapeDtypeStruct((e, dout), jnp.float32),
    )(gsum, dts2d, freq.reshape(1, -1), phase.reshape(1, -1), wrel_t, w2_t,
      b2.reshape(1, -1))


def _edge2_body(g_ref, w2_ref, b2_ref, q_ref):
    hmid = jnp.maximum(g_ref[...], 0.0)
    p = jnp.tanh(jnp.dot(hmid.astype(jnp.bfloat16),
                         w2_ref[...].astype(jnp.bfloat16),
                         preferred_element_type=jnp.float32) + b2_ref[...])
    q_ref[...] = 2.0 * p - 1.0


def _edge2(gsum, w2_t, b2):
    e, hdim = gsum.shape
    dout = w2_t.shape[1]
    grid = e // _BE
    return pl.pallas_call(
        _edge2_body,
        grid=(grid,),
        in_specs=[
            pl.BlockSpec((_BE, hdim), lambda i: (i, 0)),
            pl.BlockSpec((hdim, dout), lambda i: (0, 0)),
            pl.BlockSpec((1, dout), lambda i: (0, 0)),
        ],
        out_specs=pl.BlockSpec((_BE, dout), lambda i: (i, 0)),
        out_shape=jax.ShapeDtypeStruct((e, dout), jnp.float32),
    )(gsum, w2_t, b2.reshape(1, -1))


_NC = 2
_NS = 16
_NW = _NC * _NS
_IB = 100


def _sc_mesh():
    return plsc.VectorSubcoreMesh(core_axis_name="c", subcore_axis_name="s",
                                  num_cores=_NC, num_subcores=_NS)


_SC_PARAMS = pltpu.CompilerParams(use_tc_tiling_on_sc=False)


def _batches(ew, ch):
    out, o = [], 0
    while o < ew:
        b = min(ch, ew - o)
        out.append((o, b))
        o += b
    return out


_SB = 624


def _fill_zbuf(zbuf, dh, zr):
    def zrow(r, _):
        for t in range(dh // 16):
            zbuf[r, pl.ds(t * 16, 16)] = jnp.zeros((16,), jnp.float32)
        return 0

    lax.fori_loop(0, zr, zrow, 0)


def _zero_stripe(zbuf, acc, sid, n, zr):
    start = sid * _SB
    for zo, zb in _batches(_SB, zr):
        pltpu.sync_copy(zbuf.at[pl.ds(0, zb)], acc.at[pl.ds(start + zo, zb)])
    tail = n - _NS * _SB

    @pl.when(sid == _NS - 1)
    def _():
        pltpu.sync_copy(zbuf.at[pl.ds(0, tail)], acc.at[pl.ds(_NS * _SB, tail)])


def _copy_out(acc, out_hbm, cid, sid, n):
    start = sid * _SB
    pltpu.sync_copy(acc.at[pl.ds(start, _SB)],
                    out_hbm.at[pl.ds(cid * n + start, _SB)])
    tail = n - _NS * _SB

    @pl.when(sid == _NS - 1)
    def _():
        pltpu.sync_copy(acc.at[pl.ds(_NS * _SB, tail)],
                        out_hbm.at[pl.ds(cid * n + _NS * _SB, tail)])


def _gather_sum(a_tbl, b_tbl, dst1, src1, out_dh=None):
    n, dh = a_tbl.shape
    out_dh = out_dh or dh
    e = dst1.shape[0]
    ew = e // _NW
    ch = 160

    @functools.partial(
        pl.kernel, mesh=_sc_mesh(),
        out_type=jax.ShapeDtypeStruct((e, out_dh), jnp.float32),
        compiler_params=_SC_PARAMS,
        scratch_types=[
            pltpu.VMEM((ew,), jnp.int32),
            pltpu.VMEM((ew,), jnp.int32),
            pltpu.VMEM((ch, dh), jnp.float32),
            pltpu.VMEM((ch, dh), jnp.float32),
            pltpu.VMEM((ch, dh), jnp.float32),
            pltpu.VMEM((ch, dh), jnp.float32),
            pltpu.SemaphoreType.DMA,
            pltpu.SemaphoreType.DMA,
            pltpu.SemaphoreType.DMA,
            pltpu.SemaphoreType.DMA,
        ],
    )
    def k(a_hbm, b_hbm, dst_hbm, src_hbm, out_hbm, idxa, idxb,
          ba0, bb0, ba1, bb1, sg0, sg1, so0, so1):
        wid = lax.axis_index("s") * _NC + lax.axis_index("c")
        pltpu.sync_copy(dst_hbm.at[pl.ds(wid * ew, ew)], idxa)
        pltpu.sync_copy(src_hbm.at[pl.ds(wid * ew, ew)], idxb)

        ba = (ba0, ba1)
        bb = (bb0, bb1)
        sg = (sg0, sg1)
        so = (so0, so1)
        bs = _batches(ew, ch)
        gcps = [None, None]
        ocps = [None, None]

        def start_gather(i):
            o, ne = bs[i]
            s = i % 2
            isl = pl.ds(o, ne)
            esl = pl.ds(0, ne)
            ca = pltpu.async_copy(a_hbm.at[idxa.at[isl]], ba[s].at[esl], sg[s])
            cb = pltpu.async_copy(b_hbm.at[idxb.at[isl]], bb[s].at[esl], sg[s])
            gcps[s] = (ca, cb)

        start_gather(0)
        for i, (o, ne) in enumerate(bs):
            s = i % 2
            ca, cb = gcps[s]
            ca.wait()
            cb.wait()
            if i + 1 < len(bs):
                if ocps[1 - s] is not None:
                    ocps[1 - s].wait()
                    ocps[1 - s] = None
                start_gather(i + 1)

            def row(r, _):
                for t in range(dh // 16):
                    sl = pl.ds(t * 16, 16)
                    ba[s][r, sl] = ba[s][r, sl] + bb[s][r, sl]
                return 0

            lax.fori_loop(0, ne, row, 0)
            esl = pl.ds(0, ne)
            rsl = pl.ds(wid * ew + o, ne)
            if out_dh == dh:
                ocps[s] = pltpu.async_copy(ba[s].at[esl], out_hbm.at[rsl],
                                           so[s])
            else:
                ocps[s] = pltpu.async_copy(ba[s].at[esl],
                                           out_hbm.at[rsl, pl.ds(0, dh)],
                                           so[s])
        for s in range(2):
            if ocps[s] is not None:
                ocps[s].wait()

    return k(a_tbl, b_tbl, dst1, src1)


def _sc_scatter(q, dst1, n, dst_full=None, cs=8):
    e, dh = q.shape
    ew = e // _NW
    zr = 208
    ch = 1600
    cw = dh // cs

    outs = [jax.ShapeDtypeStruct((_NC * n, dh), jnp.float32)]
    scr = [
        pltpu.VMEM((ew,), jnp.int32),
        pltpu.VMEM((ch, cw), jnp.float32),
        pltpu.VMEM((ch, cw), jnp.float32),
        pltpu.VMEM((zr, cw), jnp.float32),
        pltpu.VMEM_SHARED((n, cw), jnp.float32),
        pltpu.SemaphoreType.DMA,
        pltpu.SemaphoreType.DMA,
    ]
    ewf = None
    if dst_full is not None:
        ewf = dst_full.shape[0] // _NW
        outs.append(jax.ShapeDtypeStruct((_NC * n, cw), jnp.float32))
        scr.append(pltpu.VMEM((ewf,), jnp.int32))

    @functools.partial(
        pl.kernel, mesh=_sc_mesh(),
        out_type=tuple(outs) if len(outs) > 1 else outs[0],
        compiler_params=_SC_PARAMS,
        scratch_types=scr,
    )
    def k(*refs):
        if dst_full is None:
            (q_hbm, dst_hbm, out_hbm, idx, qb0, qb1, zbuf, acc,
             sem0, sem1) = refs
            dstf_hbm = cnt_hbm = idxf = None
        else:
            (q_hbm, dst_hbm, dstf_hbm, out_hbm, cnt_hbm, idx, qb0, qb1, zbuf,
             acc, sem0, sem1, idxf) = refs
        cid = lax.axis_index("c")
        sid = lax.axis_index("s")
        wid = sid * _NC + cid
        qb = (qb0, qb1)
        sem = (sem0, sem1)
        pltpu.sync_copy(dst_hbm.at[pl.ds(wid * ew, ew)], idx)
        _fill_zbuf(zbuf, cw, zr)
        for p in range(cs):
            csl = pl.ds(p * cw, cw)
            _zero_stripe(zbuf, acc, sid, n, zr)
            plsc.subcore_barrier()

            bs = _batches(ew, ch)
            cps = [None, None]

            def load(i):
                o, ne = bs[i]
                s = i % 2
                cps[s] = pltpu.async_copy(
                    q_hbm.at[pl.ds(wid * ew + o, ne), csl],
                    qb[s].at[pl.ds(0, ne)], sem[s])

            load(0)
            for i, (o, ne) in enumerate(bs):
                s = i % 2
                cps[s].wait()
                if i + 1 < len(bs):
                    load(i + 1)
                pltpu.sync_copy(qb[s].at[pl.ds(0, ne)],
                                acc.at[idx.at[pl.ds(o, ne)]], add=True)
            plsc.subcore_barrier()
            start = sid * _SB
            pltpu.sync_copy(acc.at[pl.ds(start, _SB)],
                            out_hbm.at[pl.ds(cid * n + start, _SB), csl])
            tail = n - _NS * _SB

            @pl.when(sid == _NS - 1)
            def _():
                pltpu.sync_copy(acc.at[pl.ds(_NS * _SB, tail)],
                                out_hbm.at[pl.ds(cid * n + _NS * _SB, tail),
                                           csl])
            plsc.subcore_barrier()

        if dst_full is not None:
            pltpu.sync_copy(dstf_hbm.at[pl.ds(wid * ewf, ewf)], idxf)

            def fill(r, _):
                for t in range(cw // 16):
                    qb0[r, pl.ds(t * 16, 16)] = jnp.ones((16,), jnp.float32)
                return 0

            lax.fori_loop(0, ch, fill, 0)
            _zero_stripe(zbuf, acc, sid, n, zr)
            plsc.subcore_barrier()
            for o, ne in _batches(ewf, ch):
                pltpu.sync_copy(qb0.at[pl.ds(0, ne)],
                                acc.at[idxf.at[pl.ds(o, ne)]], add=True)
            plsc.subcore_barrier()
            _copy_out(acc, cnt_hbm, cid, sid, n)

    if dst_full is None:
        return k(q, dst1)
    return k(q, dst1, dst_full)


def kernel(x, dts, params, edge_index):
    src = edge_index[0]
    dst = edge_index[1]
    n, d = x.shape
    e = dst.shape[0]
    t = params['basis_freq'].shape[0]

    w1 = params['tmp_W1']
    hid = w1.shape[0]
    hpad = -hid % 128
    w1d_t = w1[:, :d].T
    w1s_t = w1[:, d:2 * d].T
    w1rel_t = jnp.pad(w1[:, 2 * d:].T, ((0, 0), (0, hpad)))
    tmp_b1 = params['tmp_b1']
    tmp_w2_t = jnp.pad(params['tmp_W2'].T, ((0, hpad), (0, 0)))

    eh = e // 2
    dts2d = dts.reshape(-1, 1)
    hv = []
    for i in range(2):
        sl = slice(i * eh, (i + 1) * eh)
        hv.append((dst[sl], src[sl], dts2d[sl]))

    a1, b1t = _node_pre(x, w1d_t, w1s_t, tmp_b1)
    s1 = []
    cnt = None
    for i, (d3, s3_, dt) in enumerate(hv):
        g = _gather_sum(a1, b1t, d3, s3_, out_dh=hid + hpad)
        q = _edge1(g, dt, params['basis_freq'], params['phase'],
                   w1rel_t, tmp_w2_t, params['tmp_b2'], hid)
        if i == 0:
            s, cnt = _sc_scatter(q, d3, n, dst_full=dst)
            s1.append(s)
        else:
            s1.append(_sc_scatter(q, d3, n))

    smp0, smp1 = params['smp']
    h, a2, b2t = _node1(x, s1, cnt, params['proj_W'].T, params['proj_b'],
                        smp0['W1'][:, :d].T, smp0['b1'], smp0['W1'][:, d:].T)

    s2 = []
    for d3, s3_, _ in hv:
        g = _gather_sum(a2, b2t, d3, s3_)
        q = _edge2(g, smp0['W2'].T, smp0['b2'])
        s2.append(_sc_scatter(q, d3, n))
    h, a3, b3t = _node2(h, s2, cnt, smp0['bn_g'], smp0['bn_b'],
                        smp1['W1'][:, :d].T, smp1['b1'], smp1['W1'][:, d:].T)

    s3 = []
    for d3, s3_, _ in hv:
        g = _gather_sum(a3, b3t, d3, s3_)
        q = _edge2(g, smp1['W2'].T, smp1['b2'])
        s3.append(_sc_scatter(q, d3, n))

    return _node3(h, s3, cnt, smp1['bn_g'], smp1['bn_b'], params['clf'])
